# baseline (speedup 1.0000x reference)
import numpy as np
import jax
import jax.numpy as jnp
from jax.experimental import pallas as pl
from jax.experimental.pallas import tpu as pltpu

_lipsLowerInner = [78, 95, 88, 178, 87, 14, 317, 402, 318, 324, 308]
_lipsLowerOuter = [146, 91, 181, 84, 17, 314, 405, 321, 375, 291]
_lipsUpperInner = [78, 191, 80, 81, 82, 13, 312, 311, 310, 415, 308]
_lipsUpperOuter = [61, 185, 40, 39, 37, 0, 267, 269, 270, 409, 291]
_LIPS = np.array(
    _lipsUpperOuter + _lipsLowerOuter + _lipsUpperInner + _lipsLowerInner,
    dtype=np.int64,
)

_N = 32768
_W = 543 * 3
_R = 256

_LB = 1404
_RB = 1565

_r = np.arange(64)
_MASK_L = ((_r < 63) & (_r % 3 != 2)).astype(np.float32)
_MASK_R = ((_r >= 1) & ((_r - 1) % 3 != 2)).astype(np.float32)

_sel_main = np.full(236, -1, dtype=np.int64)
for _k in range(43):
    for _c in range(2):
        _sel_main[2 * _k + _c] = 3 * _LIPS[_k] + _c
for _k in range(33):
    for _c in range(2):
        _sel_main[128 + 2 * _k + _c] = 3 * (489 + _k) + _c

_OH_MAIN = np.zeros((_W, 236), dtype=np.float32)
for _j in range(236):
    if _sel_main[_j] >= 0:
        _OH_MAIN[_sel_main[_j], _j] = 1.0

_OH_L = np.zeros((64, 236), dtype=np.float32)
for _k in range(21):
    for _c in range(2):
        _OH_L[3 * _k + _c, 86 + 2 * _k + _c] = 1.0

_OH_R = np.zeros((64, 236), dtype=np.float32)
for _k in range(21):
    for _c in range(2):
        _OH_R[3 * _k + _c + 1, 194 + 2 * _k + _c] = 1.0


def _body(x_ref, ml_ref, mr_ref, ohm_ref, ohl_ref, ohr_ref,
          out_ref, s1, s2, msum_l, msq_l, msum_r, msq_r, cnt):
    i = pl.program_id(0)

    @pl.when(i == 0)
    def _init():
        s1[...] = jnp.zeros_like(s1)
        s2[...] = jnp.zeros_like(s2)
        msum_l[...] = jnp.zeros_like(msum_l)
        msq_l[...] = jnp.zeros_like(msq_l)
        msum_r[...] = jnp.zeros_like(msum_r)
        msq_r[...] = jnp.zeros_like(msq_r)
        cnt[0, 0] = 0.0
        cnt[0, 1] = 0.0

    x = x_ref[...]
    s1[...] += jnp.sum(x, axis=0)[None, :]
    s2[...] += jnp.sum(x * x, axis=0)[None, :]

    xl = x[:, _LB:_LB + 64]
    xr = x[:, _RB:_RB + 64]
    mask_l = ml_ref[...]
    mask_r = mr_ref[...]
    bad_l = jnp.any(jnp.isnan(xl) & (mask_l > 0), axis=1)
    bad_r = jnp.any(jnp.isnan(xr) & (mask_r > 0), axis=1)
    wl = jnp.where(bad_l, 0.0, 1.0)[:, None]
    wr = jnp.where(bad_r, 0.0, 1.0)[:, None]
    xlz = jnp.where(wl > 0, xl, 0.0)
    xrz = jnp.where(wr > 0, xr, 0.0)
    msum_l[...] += jnp.sum(xlz, axis=0)[None, :]
    msq_l[...] += jnp.sum(xlz * xlz, axis=0)[None, :]
    msum_r[...] += jnp.sum(xrz, axis=0)[None, :]
    msq_r[...] += jnp.sum(xrz * xrz, axis=0)[None, :]
    cnt[0, 0] += jnp.sum(wl)
    cnt[0, 1] += jnp.sum(wr)

    @pl.when(i == (_N // _R) - 1)
    def _finish():
        n = jnp.float32(_N)
        mean_f = s1[...] / n
        var_f = jnp.maximum((s2[...] - n * mean_f * mean_f) / (n - 1.0), 0.0)
        std_f = jnp.sqrt(var_f)

        nl = cnt[0, 0]
        nr = cnt[0, 1]
        mean_l = msum_l[...] / nl
        var_l = jnp.maximum((msq_l[...] - nl * mean_l * mean_l) / (nl - 1.0), 0.0)
        std_l = jnp.sqrt(var_l)
        mean_r = msum_r[...] / nr
        var_r = jnp.maximum((msq_r[...] - nr * mean_r * mean_r) / (nr - 1.0), 0.0)
        std_r = jnp.sqrt(var_r)

        oh_m = ohm_ref[...]
        oh_l = ohl_ref[...]
        oh_r = ohr_ref[...]
        m236 = (jnp.dot(mean_f, oh_m, preferred_element_type=jnp.float32)
                + jnp.dot(mean_l, oh_l, preferred_element_type=jnp.float32)
                + jnp.dot(mean_r, oh_r, preferred_element_type=jnp.float32))
        s236 = (jnp.dot(std_f, oh_m, preferred_element_type=jnp.float32)
                + jnp.dot(std_l, oh_l, preferred_element_type=jnp.float32)
                + jnp.dot(std_r, oh_r, preferred_element_type=jnp.float32))
        out = jnp.concatenate([m236, s236], axis=1)
        out_ref[...] = jnp.where(jnp.isnan(out), 0.0, out)


def kernel(x):
    xf = x.reshape(_N, _W)
    out = pl.pallas_call(
        _body,
        grid=(_N // _R,),
        in_specs=[
            pl.BlockSpec((_R, _W), lambda i: (i, 0)),
            pl.BlockSpec((1, 64), lambda i: (0, 0)),
            pl.BlockSpec((1, 64), lambda i: (0, 0)),
            pl.BlockSpec((_W, 236), lambda i: (0, 0)),
            pl.BlockSpec((64, 236), lambda i: (0, 0)),
            pl.BlockSpec((64, 236), lambda i: (0, 0)),
        ],
        out_specs=pl.BlockSpec((1, 472), lambda i: (0, 0)),
        out_shape=jax.ShapeDtypeStruct((1, 472), jnp.float32),
        scratch_shapes=[
            pltpu.VMEM((1, _W), jnp.float32),
            pltpu.VMEM((1, _W), jnp.float32),
            pltpu.VMEM((1, 64), jnp.float32),
            pltpu.VMEM((1, 64), jnp.float32),
            pltpu.VMEM((1, 64), jnp.float32),
            pltpu.VMEM((1, 64), jnp.float32),
            pltpu.SMEM((1, 2), jnp.float32),
        ],
    )(xf,
      jnp.asarray(_MASK_L)[None, :], jnp.asarray(_MASK_R)[None, :],
      jnp.asarray(_OH_MAIN), jnp.asarray(_OH_L), jnp.asarray(_OH_R))
    return out.reshape(472)
